# baseline (speedup 1.0000x reference)
import functools

import jax
import jax.numpy as jnp
from jax import lax
from jax.experimental import pallas as pl
from jax.experimental.pallas import tpu as pltpu
from jax.experimental.pallas import tpu_sc as plsc

D = 32
L = 16
NC = 2
NS = 16
NW = NC * NS
SUB = 128
NBUF = 2


@functools.lru_cache(maxsize=None)
def _build_score_kernel(total: int):
    per_w = total // NW
    nsub = per_w // SUB
    n_blocks = total // SUB
    mesh = plsc.VectorSubcoreMesh(core_axis_name="c", subcore_axis_name="s")

    @functools.partial(
        pl.kernel,
        mesh=mesh,
        compiler_params=pltpu.CompilerParams(needs_layout_passes=False,
                                             use_tc_tiling_on_sc=False),
        out_type=jax.ShapeDtypeStruct((n_blocks, SUB), jnp.float32),
        scratch_types=[
            pltpu.VMEM((nsub, SUB), jnp.int32),
            pltpu.VMEM((nsub, SUB), jnp.int32),
            pltpu.VMEM((nsub, SUB), jnp.int32),
            pltpu.VMEM((SUB, D), jnp.float32),
            pltpu.VMEM((SUB, D), jnp.float32),
            pltpu.VMEM((SUB, D), jnp.float32),
            pltpu.VMEM((SUB, D), jnp.float32),
            pltpu.VMEM((SUB, D), jnp.float32),
            pltpu.VMEM((SUB, D), jnp.float32),
            pltpu.VMEM((nsub, SUB), jnp.float32),
            pltpu.SemaphoreType.DMA,
            pltpu.SemaphoreType.DMA,
        ],
    )
    def score_kernel(head_hbm, tail_hbm, rel_hbm, itab_hbm, rtab_hbm,
                     out_hbm, hidx, tidx, ridx, hrows0, hrows1,
                     trows0, trows1, rrows0, rrows1, acc, sem0, sem1):
        sems = [sem0, sem1]
        hrows = [hrows0, hrows1]
        trows = [trows0, trows1]
        rrows = [rrows0, rrows1]
        wid = lax.axis_index("s") * NC + lax.axis_index("c")
        base = wid * nsub

        pltpu.sync_copy(head_hbm.at[pl.ds(base, nsub)], hidx)
        pltpu.sync_copy(tail_hbm.at[pl.ds(base, nsub)], tidx)
        pltpu.sync_copy(rel_hbm.at[pl.ds(base, nsub)], ridx)

        def start(c, b):
            pltpu.make_async_copy(itab_hbm.at[hidx.at[c]], hrows[b],
                                  sems[b]).start()
            pltpu.make_async_copy(itab_hbm.at[tidx.at[c]], trows[b],
                                  sems[b]).start()
            pltpu.make_async_copy(rtab_hbm.at[ridx.at[c]], rrows[b],
                                  sems[b]).start()

        def wait(c, b):
            pltpu.make_async_copy(itab_hbm.at[hidx.at[c]], hrows[b],
                                  sems[b]).wait()
            pltpu.make_async_copy(itab_hbm.at[tidx.at[c]], trows[b],
                                  sems[b]).wait()
            pltpu.make_async_copy(rtab_hbm.at[ridx.at[c]], rrows[b],
                                  sems[b]).wait()

        for b in range(NBUF):
            start(b, b)

        lane = lax.iota(jnp.int32, L)

        def compute(c, b):
            for g in range(SUB // L):
                outv = jnp.zeros((L,), jnp.float32)
                for l in range(L):
                    row = g * L + l
                    h0 = hrows[b][row, pl.ds(0, L)]
                    h1 = hrows[b][row, pl.ds(L, L)]
                    t0 = trows[b][row, pl.ds(0, L)]
                    t1 = trows[b][row, pl.ds(L, L)]
                    r0 = rrows[b][row, pl.ds(0, L)]
                    r1 = rrows[b][row, pl.ds(L, L)]
                    d0 = h0 + r0 - t0
                    d1 = h1 + r1 - t1
                    q = d0 * d0 + d1 * d1
                    s = jnp.sum(q)
                    outv = jnp.where(lane == l, -s, outv)
                acc[c, pl.ds(g * L, L)] = outv

        def body(i, carry):
            for b in range(NBUF):
                c = i * NBUF + b
                wait(c, b)
                compute(c, b)
                nxt = c + NBUF

                @pl.when(nxt < nsub)
                def _():
                    start(nxt, b)
            return carry

        lax.fori_loop(0, nsub // NBUF, body, 0)
        pltpu.sync_copy(acc, out_hbm.at[pl.ds(base, nsub)])

    return score_kernel


def kernel(head_ids, tail_ids, relation_ids, i_embeddings, r_embeddings):
    B, K = head_ids.shape
    total = B * K
    n_blocks = total // SUB
    h2 = head_ids.astype(jnp.int32).reshape(n_blocks, SUB)
    t2 = tail_ids.astype(jnp.int32).reshape(n_blocks, SUB)
    r2 = relation_ids.astype(jnp.int32).reshape(n_blocks, SUB)
    score = _build_score_kernel(total)
    out = score(h2, t2, r2, i_embeddings, r_embeddings)
    prediction = out.reshape(B, K)
    pos_pred = prediction[:, :2].reshape(-1)
    neg_pred = prediction[:, 2:].reshape(-1)
    return prediction, pos_pred, neg_pred
